# baseline (speedup 1.0000x reference)
import functools

import numpy as np

import jax
import jax.numpy as jnp
from jax import lax
from jax.experimental import pallas as pl
from jax.experimental.pallas import tpu as pltpu
from jax.experimental.pallas import tpu_sc as plsc

B = 4096
FIELD = 26
VOCAB = 100000
EMB = 16
H = 32
N = B * FIELD
NC, NS = 2, 16
NW = NC * NS
PER_W = N // NW
CHUNK = 128
NCH = PER_W // CHUNK

_E_NP = np.repeat(np.eye(FIELD, dtype=np.float32), EMB, axis=1)


def _sc_gather(flat_idx, t2, t1):
    mesh = plsc.VectorSubcoreMesh(core_axis_name="c", subcore_axis_name="s")

    @functools.partial(
        pl.kernel,
        out_type=(
            jax.ShapeDtypeStruct((N, EMB), jnp.float32),
            jax.ShapeDtypeStruct((N,), jnp.float32),
        ),
        mesh=mesh,
        scratch_types=[
            pltpu.VMEM((NCH, CHUNK), jnp.int32),
            pltpu.VMEM((PER_W, EMB), jnp.float32),
            pltpu.VMEM((PER_W,), jnp.float32),
            pltpu.SemaphoreType.DMA,
            pltpu.SemaphoreType.DMA,
        ],
        compiler_params=pltpu.CompilerParams(use_tc_tiling_on_sc=False),
    )
    def k(idx_hbm, t2_hbm, t1_hbm, out2_hbm, out1_hbm,
          idx_v, rows2_v, rows1_v, sem2, sem1):
        wid = lax.axis_index("s") * NC + lax.axis_index("c")
        base = wid * PER_W
        pltpu.sync_copy(idx_hbm.at[wid], idx_v)

        def fire(j, carry):
            pltpu.async_copy(
                t2_hbm.at[idx_v.at[j]], rows2_v.at[pl.ds(j * CHUNK, CHUNK)],
                sem2)
            pltpu.async_copy(
                t1_hbm.at[idx_v.at[j]], rows1_v.at[pl.ds(j * CHUNK, CHUNK)],
                sem1)
            return carry

        lax.fori_loop(0, NCH, fire, 0)
        pltpu.make_async_copy(t2_hbm.at[pl.ds(0, PER_W)], rows2_v, sem2).wait()
        pltpu.make_async_copy(t1_hbm.at[pl.ds(0, PER_W)], rows1_v, sem1).wait()
        pltpu.sync_copy(rows2_v, out2_hbm.at[pl.ds(base, PER_W)])
        pltpu.sync_copy(rows1_v, out1_hbm.at[pl.ds(base, PER_W)])

    return k(flat_idx, t2, t1)


def _mlp(g1, g2, xv, e_mat, w1f, w1s, c1, w2, b2, w3, b3):
    blk = 512
    d2 = FIELD * EMB

    def body(g1_ref, g2_ref, xv_ref, e_ref, w1f_ref, w1s_ref, c1_ref,
             w2_ref, b2_ref, w3_ref, b3_ref, out_ref):
        xv_b = xv_ref[...]
        ff = g1_ref[...] * xv_b
        xv16 = jnp.dot(xv_b, e_ref[...], preferred_element_type=jnp.float32)
        fs = g2_ref[...] * xv16
        h = jnp.tanh(
            jnp.dot(ff, w1f_ref[...], preferred_element_type=jnp.float32)
            + jnp.dot(fs, w1s_ref[...], preferred_element_type=jnp.float32)
            + c1_ref[...])
        h = jnp.tanh(
            jnp.dot(h, w2_ref[...], preferred_element_type=jnp.float32)
            + b2_ref[...])
        out_ref[...] = (
            jnp.dot(h, w3_ref[...], preferred_element_type=jnp.float32)
            + b3_ref[...])

    out = pl.pallas_call(
        body,
        grid=(B // blk,),
        in_specs=[
            pl.BlockSpec((blk, FIELD), lambda i: (i, 0)),
            pl.BlockSpec((blk, d2), lambda i: (i, 0)),
            pl.BlockSpec((blk, FIELD), lambda i: (i, 0)),
            pl.BlockSpec((FIELD, d2), lambda i: (0, 0)),
            pl.BlockSpec((FIELD, H), lambda i: (0, 0)),
            pl.BlockSpec((d2, H), lambda i: (0, 0)),
            pl.BlockSpec((1, H), lambda i: (0, 0)),
            pl.BlockSpec((H, H), lambda i: (0, 0)),
            pl.BlockSpec((1, H), lambda i: (0, 0)),
            pl.BlockSpec((H, 1), lambda i: (0, 0)),
            pl.BlockSpec((1, 1), lambda i: (0, 0)),
        ],
        out_specs=pl.BlockSpec((blk, 1), lambda i: (i, 0)),
        out_shape=jax.ShapeDtypeStruct((B, 1), jnp.float32),
    )(g1, g2, xv, e_mat, w1f, w1s, c1, w2, b2, w3, b3)
    return out[:, 0]


def kernel(Xi, Xv, fm_bias, first_tables, second_tables, W1, b1, W2, b2, W3, b3):
    idx = Xi[:, :, 0]
    offs = (jnp.arange(FIELD, dtype=jnp.int32) * VOCAB)[None, :]
    flat_idx = (idx + offs).reshape(NW, NCH, CHUNK)
    t2 = second_tables.reshape(FIELD * VOCAB, EMB)
    t1 = first_tables.reshape(FIELD * VOCAB)
    rows2, rows1 = _sc_gather(flat_idx, t2, t1)
    g2 = rows2.reshape(B, FIELD * EMB)
    g1 = rows1.reshape(B, FIELD)
    e_mat = jnp.asarray(_E_NP)
    c1 = (fm_bias[0] * W1[0] + b1)[None, :]
    return _mlp(g1, g2, Xv, e_mat, W1[1:1 + FIELD], W1[1 + FIELD:], c1,
                W2, b2[None, :], W3, b3[None, :])
